# baseline (speedup 1.0000x reference)
import math

import jax
import jax.numpy as jnp
from jax import lax
from jax.experimental import pallas as pl

_F32 = jnp.float32
_NEG = -9e15
_ALPHA_GAT = 0.01
_ALPHA = 0.1
_LAMDA = 0.5
_N = 1024


def _lrelu(x, a):
    return jnp.where(x >= 0, x, a * x)


def _elu(x):
    return jnp.where(x > 0, x, jnp.exp(x) - 1.0)


def _dot(a, b):
    return jnp.dot(a, b, preferred_element_type=_F32,
                   precision=lax.Precision.HIGHEST)


def _dot_nt(a, b):
    return lax.dot_general(a, b, (((1,), (1,)), ((), ())),
                           preferred_element_type=_F32,
                           precision=lax.Precision.HIGHEST)


def _attention(adj6, wh, av, nheads, feat, bm=256):
    n = _N
    wb = wh.shape[0]
    nrb = n // bm

    if wb == 6:
        wh_idx = lambda bt, h, i: (bt, 0, h)
        whr_idx = lambda bt, h, i: (bt, i, h)
    else:
        wh_idx = lambda bt, h, i: (bt // 3, 0, h)
        whr_idx = lambda bt, h, i: (bt // 3, i, h)

    def body(adj_ref, wh_ref, whr_ref, av_ref, out_ref):
        whf = wh_ref[0]
        av_ = av_ref[0, 0]
        a_lo = av_[:feat].reshape(feat, 1)
        a_hi = av_[feat:].reshape(1, feat)
        f2 = _dot_nt(a_hi, whf)
        f1 = _dot(whr_ref[0], a_lo)
        e = _lrelu(f1 + f2, _ALPHA_GAT)
        e = jnp.where(adj_ref[0] > 0, e, _NEG)
        m = jnp.max(e, axis=1, keepdims=True)
        p = jnp.exp(e - m)
        att = p / jnp.sum(p, axis=1, keepdims=True)
        out_ref[0] = _elu(_dot(att, whf))

    return pl.pallas_call(
        body,
        grid=(6, nheads, nrb),
        in_specs=[
            pl.BlockSpec((1, bm, n), lambda bt, h, i: (bt, i, 0)),
            pl.BlockSpec((1, n, feat), wh_idx),
            pl.BlockSpec((1, bm, feat), whr_idx),
            pl.BlockSpec((1, 1, 2 * feat),
                         lambda bt, h, i: ((bt // 3) * nheads + h, 0, 0)),
        ],
        out_specs=pl.BlockSpec((1, bm, feat), lambda bt, h, i: (bt, i, h)),
        out_shape=jax.ShapeDtypeStruct((6, n, nheads * feat), _F32),
    )(adj6, wh, wh, av)


def _bmm(a, b, a_idx, b_idx, gb, bm=256, bn=1024, extras=(), epilogue=None):
    M, K = a.shape[-2:]
    N = b.shape[-1]
    bn = min(bn, N)
    nm, nn = M // bm, N // bn

    def body(a_ref, b_ref, *rest):
        out_ref = rest[-1]
        ex = [r[0] for r in rest[:-1]]
        r = _dot(a_ref[0], b_ref[0])
        out_ref[0] = epilogue(r, *ex) if epilogue is not None else r

    in_specs = [
        pl.BlockSpec((1, bm, K), lambda g, m, n, f=a_idx: (f(g), m, 0)),
        pl.BlockSpec((1, K, bn), lambda g, m, n, f=b_idx: (f(g), 0, n)),
    ]
    for (_, blk, idx) in extras:
        in_specs.append(pl.BlockSpec(blk, idx))

    return pl.pallas_call(
        body,
        grid=(gb, nm, nn),
        in_specs=in_specs,
        out_specs=pl.BlockSpec((1, bm, bn), lambda g, m, n: (g, m, n)),
        out_shape=jax.ShapeDtypeStruct((gb, M, N), _F32),
    )(a, b, *[e[0] for e in extras])


def _mean3(xs, bm=256):
    n, f = xs.shape[-2:]

    def body(x0, x1, x2, o):
        o[0] = (x0[0] + x1[0] + x2[0]) / 3.0

    return pl.pallas_call(
        body,
        grid=(2, n // bm),
        in_specs=[
            pl.BlockSpec((1, bm, f), lambda g, i, k=k: (3 * g + k, i, 0))
            for k in range(3)
        ],
        out_specs=pl.BlockSpec((1, bm, f), lambda g, i: (g, i, 0)),
        out_shape=jax.ShapeDtypeStruct((2, n, f), _F32),
    )(xs, xs, xs)


def _viewmeans(xs, bm=256):
    n, f = xs.shape[-2:]
    scale = 1.0 / (n * f)

    def body(x_ref, o_ref):
        i = pl.program_id(1)

        @pl.when(i == 0)
        def _():
            o_ref[0] = jnp.zeros((1, 128), _F32)

        s = jnp.sum(x_ref[0]) * scale
        o_ref[0] += jnp.full((1, 128), s, _F32)

    return pl.pallas_call(
        body,
        grid=(6, n // bm),
        in_specs=[pl.BlockSpec((1, bm, f), lambda g, i: (g, i, 0))],
        out_specs=pl.BlockSpec((1, 1, 128), lambda g, i: (g, 0, 0)),
        out_shape=jax.ShapeDtypeStruct((6, 1, 128), _F32),
    )(xs)


def _view_mlp(mu, w1, b1, w2, b2):
    def body(mu_ref, w1_ref, b1_ref, w2_ref, b2_ref, o_ref):
        h = jnp.maximum(_dot_nt(mu_ref[0], w1_ref[0]) + b1_ref[0], 0.0)
        o = _dot_nt(h, w2_ref[0]) + b2_ref[0]
        o_ref[0] = 1.0 / (1.0 + jnp.exp(-o))

    vg, hid = w1.shape[2], w1.shape[1]
    return pl.pallas_call(
        body,
        grid=(2,),
        in_specs=[
            pl.BlockSpec((1, 1, vg), lambda g: (g, 0, 0)),
            pl.BlockSpec((1, hid, vg), lambda g: (g, 0, 0)),
            pl.BlockSpec((1, 1, hid), lambda g: (g, 0, 0)),
            pl.BlockSpec((1, vg, hid), lambda g: (g, 0, 0)),
            pl.BlockSpec((1, 1, vg), lambda g: (g, 0, 0)),
        ],
        out_specs=pl.BlockSpec((1, 1, vg), lambda g: (g, 0, 0)),
        out_shape=jax.ShapeDtypeStruct((2, 1, vg), _F32),
    )(mu, w1, b1, w2, b2)


def _cnn(xs1, xs2, attw, w, bias, bm=256):
    n, f = xs1.shape[-2:]
    outc = w.shape[-1]

    def body(x10, x11, x12, x20, x21, x22, a_ref, w_ref, b_ref, o_ref):
        a = a_ref[0, 0]
        views = (x10, x11, x12, x20, x21, x22)
        acc = jnp.zeros((bm, outc), _F32)
        for c in range(6):
            xb = jnp.maximum(views[c][0], 0.0)
            acc += a[c] * _dot(xb, w_ref[0, c])
        o_ref[0] = acc + b_ref[0]

    in_specs = [
        pl.BlockSpec((1, bm, f), lambda g, i, k=k: (3 * g + k, i, 0))
        for k in range(3)
    ] * 2
    in_specs += [
        pl.BlockSpec((1, 1, 6), lambda g, i: (g, 0, 0)),
        pl.BlockSpec((1, 6, f, outc), lambda g, i: (g, 0, 0, 0)),
        pl.BlockSpec((1, 1, outc), lambda g, i: (g, 0, 0)),
    ]
    return pl.pallas_call(
        body,
        grid=(2, n // bm),
        in_specs=in_specs,
        out_specs=pl.BlockSpec((1, bm, outc), lambda g, i: (g, i, 0)),
        out_shape=jax.ShapeDtypeStruct((2, n, outc), _F32),
    )(xs1, xs1, xs1, xs2, xs2, xs2, attw, w, bias)


def _final(emb, bm=256):
    n, outc = emb.shape[-2:]

    def body(a_ref, b_ref, o_ref):
        o_ref[...] = _dot_nt(a_ref[0], b_ref[0])

    return pl.pallas_call(
        body,
        grid=(n // bm,),
        in_specs=[
            pl.BlockSpec((1, bm, outc), lambda i: (0, i, 0)),
            pl.BlockSpec((1, n, outc), lambda i: (1, 0, 0)),
        ],
        out_specs=pl.BlockSpec((bm, n), lambda i: (i, 0)),
        out_shape=jax.ShapeDtypeStruct((n, n), _F32),
    )(emb, emb)


def kernel(mi_feature, d_feature, mm_g, mm_s, mm_h, dd_g, dd_s, dd_h,
           params_x, params_y):
    px, py = params_x, params_y
    x0 = jnp.stack([mi_feature.T, d_feature.T])
    adj6 = jnp.stack([mm_g, mm_s, mm_h, dd_g, dd_s, dd_h])

    wcat1 = jnp.stack([
        jnp.concatenate([w for (w, _) in p["gat1"]["heads"]], axis=1)
        for p in (px, py)])
    av1 = jnp.stack([a[:, 0] for p in (px, py)
                     for (_, a) in p["gat1"]["heads"]])[:, None, :]
    wout1 = jnp.stack([p["gat1"]["out"][0] for p in (px, py)])
    avo1 = jnp.stack([p["gat1"]["out"][1][:, 0] for p in (px, py)])[:, None, :]
    wcat2 = jnp.stack([
        jnp.concatenate([w for (w, _) in p["gat2"]["heads"]], axis=1)
        for p in (px, py)])
    av2 = jnp.stack([a[:, 0] for p in (px, py)
                     for (_, a) in p["gat2"]["heads"]])[:, None, :]
    wout2 = jnp.stack([p["gat2"]["out"][0] for p in (px, py)])
    avo2 = jnp.stack([p["gat2"]["out"][1][:, 0] for p in (px, py)])[:, None, :]

    ident = lambda g: g
    per_branch = lambda g: g // 3

    wh1 = _bmm(x0, wcat1, ident, ident, gb=2)
    hp1 = _attention(adj6, wh1, av1, nheads=2, feat=1024)
    whb = _bmm(hp1, wout1, ident, per_branch, gb=6)
    hpb = _attention(adj6, whb, avo1, nheads=1, feat=512)
    wh2 = _bmm(hpb, wcat2, ident, per_branch, gb=6)
    hp2 = _attention(adj6, wh2, av2, nheads=2, feat=512)
    whd = _bmm(hp2, wout2, ident, per_branch, gb=6)
    a_adj = _attention(adj6, whd, avo2, nheads=1, feat=1024)

    fc0wt = jnp.stack([p["fc0_W"].T for p in (px, py)])
    fc0b = jnp.stack([p["fc0_b"] for p in (px, py)])[:, None, :]
    li = _bmm(x0, fc0wt, ident, ident, gb=2,
              extras=[(fc0b, (1, 1, 1024), lambda g, m, n: (g, 0, n))],
              epilogue=lambda r, b: _lrelu(r + b, 0.25))
    h0 = li
    xs_layers = []
    for l in (1, 2):
        w_l = jnp.stack([p["convs"][l - 1] for p in (px, py)])
        theta = math.log(_LAMDA / l + 1.0)
        sup = _bmm(a_adj, li, ident, per_branch, gb=6,
                   extras=[(h0, (1, 256, 1024),
                            lambda g, m, n: (g // 3, m, n))],
                   epilogue=lambda r, h0b: (1.0 - _ALPHA) * r + _ALPHA * h0b)
        xs_l = _bmm(w_l, sup, per_branch, ident, gb=6, bn=512,
                    extras=[(sup, (1, 256, 512),
                             lambda g, m, n: (g, m, n)),
                            (li, (1, 256, 512),
                             lambda g, m, n: (g // 3, m, n))],
                    epilogue=lambda r, s_, l_, th=theta:
                        _lrelu(th * r + (1.0 - th) * s_ + l_, 0.25))
        xs_layers.append(xs_l)
        li = _mean3(xs_l)

    mu1 = _viewmeans(xs_layers[0])[:, 0, 0]
    mu2 = _viewmeans(xs_layers[1])[:, 0, 0]
    mu = jnp.concatenate([mu1.reshape(2, 3), mu2.reshape(2, 3)],
                         axis=1).reshape(2, 1, 6)
    fc1w = jnp.stack([p["fc1_W"] for p in (px, py)])
    fc1b = jnp.stack([p["fc1_b"] for p in (px, py)])[:, None, :]
    fc2w = jnp.stack([p["fc2_W"] for p in (px, py)])
    fc2b = jnp.stack([p["fc2_b"] for p in (px, py)])[:, None, :]
    attw = _view_mlp(mu, fc1w, fc1b, fc2w, fc2b)

    cnnw = jnp.stack([jnp.transpose(p["cnn_W"][..., 0], (1, 2, 0))
                      for p in (px, py)])
    cnnb = jnp.stack([p["cnn_b"] for p in (px, py)])[:, None, :]
    emb = _cnn(xs_layers[0], xs_layers[1], attw, cnnw, cnnb)
    return _final(emb)


# speedup vs baseline: 1.9889x; 1.9889x over previous
import math

import jax
import jax.numpy as jnp
from jax import lax
from jax.experimental import pallas as pl

_F32 = jnp.float32
_NEG = -9e15
_ALPHA_GAT = 0.01
_ALPHA = 0.1
_LAMDA = 0.5
_N = 1024


def _lrelu(x, a):
    return jnp.where(x >= 0, x, a * x)


def _elu(x):
    return jnp.where(x > 0, x, jnp.exp(x) - 1.0)


def _dot(a, b):
    return jnp.dot(a, b, preferred_element_type=_F32)


def _dot_nt(a, b):
    return lax.dot_general(a, b, (((1,), (1,)), ((), ())),
                           preferred_element_type=_F32)


def _attention(adj6, wh, av, nheads, feat, bm=256):
    n = _N
    wb = wh.shape[0]
    nrb = n // bm

    if wb == 6:
        wh_idx = lambda bt, h, i: (bt, 0, h)
        whr_idx = lambda bt, h, i: (bt, i, h)
    else:
        wh_idx = lambda bt, h, i: (bt // 3, 0, h)
        whr_idx = lambda bt, h, i: (bt // 3, i, h)

    def body(adj_ref, wh_ref, whr_ref, av_ref, out_ref):
        whf = wh_ref[0]
        av_ = av_ref[0, 0]
        bf = jnp.bfloat16
        a_lo = av_[:feat].reshape(feat, 1).astype(bf)
        a_hi = av_[feat:].reshape(feat, 1).astype(bf)
        f2 = jnp.reshape(_dot(whf.astype(bf), a_hi), (1, n))
        f1 = _dot(whr_ref[0].astype(bf), a_lo)
        e = _lrelu(f1 + f2, _ALPHA_GAT)
        e = jnp.where(adj_ref[0] > 0, e, _NEG)
        m = jnp.max(e, axis=1, keepdims=True)
        p = jnp.exp(e - m)
        att = p / jnp.sum(p, axis=1, keepdims=True)
        out_ref[0] = _elu(_dot(att, whf))

    return pl.pallas_call(
        body,
        grid=(6, nheads, nrb),
        in_specs=[
            pl.BlockSpec((1, bm, n), lambda bt, h, i: (bt, i, 0)),
            pl.BlockSpec((1, n, feat), wh_idx),
            pl.BlockSpec((1, bm, feat), whr_idx),
            pl.BlockSpec((1, 1, 2 * feat),
                         lambda bt, h, i: ((bt // 3) * nheads + h, 0, 0)),
        ],
        out_specs=pl.BlockSpec((1, bm, feat), lambda bt, h, i: (bt, i, h)),
        out_shape=jax.ShapeDtypeStruct((6, n, nheads * feat), _F32),
    )(adj6, wh, wh, av)


def _bmm(a, b, a_idx, b_idx, gb, bm=256, bn=1024, extras=(), epilogue=None):
    M, K = a.shape[-2:]
    N = b.shape[-1]
    bn = min(bn, N)
    nm, nn = M // bm, N // bn

    def body(a_ref, b_ref, *rest):
        out_ref = rest[-1]
        ex = [r[0] for r in rest[:-1]]
        r = _dot(a_ref[0], b_ref[0])
        out_ref[0] = epilogue(r, *ex) if epilogue is not None else r

    in_specs = [
        pl.BlockSpec((1, bm, K), lambda g, m, n, f=a_idx: (f(g), m, 0)),
        pl.BlockSpec((1, K, bn), lambda g, m, n, f=b_idx: (f(g), 0, n)),
    ]
    for (_, blk, idx) in extras:
        in_specs.append(pl.BlockSpec(blk, idx))

    return pl.pallas_call(
        body,
        grid=(gb, nm, nn),
        in_specs=in_specs,
        out_specs=pl.BlockSpec((1, bm, bn), lambda g, m, n: (g, m, n)),
        out_shape=jax.ShapeDtypeStruct((gb, M, N), _F32),
    )(a, b, *[e[0] for e in extras])


def _mean3(xs, bm=256):
    n, f = xs.shape[-2:]

    def body(x0, x1, x2, o):
        o[0] = (x0[0] + x1[0] + x2[0]) / 3.0

    return pl.pallas_call(
        body,
        grid=(2, n // bm),
        in_specs=[
            pl.BlockSpec((1, bm, f), lambda g, i, k=k: (3 * g + k, i, 0))
            for k in range(3)
        ],
        out_specs=pl.BlockSpec((1, bm, f), lambda g, i: (g, i, 0)),
        out_shape=jax.ShapeDtypeStruct((2, n, f), _F32),
    )(xs, xs, xs)


def _viewmeans(xs, bm=256):
    n, f = xs.shape[-2:]
    scale = 1.0 / (n * f)

    def body(x_ref, o_ref):
        i = pl.program_id(1)

        @pl.when(i == 0)
        def _():
            o_ref[0] = jnp.zeros((1, 128), _F32)

        s = jnp.sum(x_ref[0]) * scale
        o_ref[0] += jnp.full((1, 128), s, _F32)

    return pl.pallas_call(
        body,
        grid=(6, n // bm),
        in_specs=[pl.BlockSpec((1, bm, f), lambda g, i: (g, i, 0))],
        out_specs=pl.BlockSpec((1, 1, 128), lambda g, i: (g, 0, 0)),
        out_shape=jax.ShapeDtypeStruct((6, 1, 128), _F32),
    )(xs)


def _view_mlp(mu, w1, b1, w2, b2):
    def body(mu_ref, w1_ref, b1_ref, w2_ref, b2_ref, o_ref):
        h = jnp.maximum(_dot_nt(mu_ref[0], w1_ref[0]) + b1_ref[0], 0.0)
        o = _dot_nt(h, w2_ref[0]) + b2_ref[0]
        o_ref[0] = 1.0 / (1.0 + jnp.exp(-o))

    vg, hid = w1.shape[2], w1.shape[1]
    return pl.pallas_call(
        body,
        grid=(2,),
        in_specs=[
            pl.BlockSpec((1, 1, vg), lambda g: (g, 0, 0)),
            pl.BlockSpec((1, hid, vg), lambda g: (g, 0, 0)),
            pl.BlockSpec((1, 1, hid), lambda g: (g, 0, 0)),
            pl.BlockSpec((1, vg, hid), lambda g: (g, 0, 0)),
            pl.BlockSpec((1, 1, vg), lambda g: (g, 0, 0)),
        ],
        out_specs=pl.BlockSpec((1, 1, vg), lambda g: (g, 0, 0)),
        out_shape=jax.ShapeDtypeStruct((2, 1, vg), _F32),
    )(mu, w1, b1, w2, b2)


def _cnn(xs1, xs2, attw, w, bias, bm=256):
    n, f = xs1.shape[-2:]
    outc = w.shape[-1]

    def body(x10, x11, x12, x20, x21, x22, a_ref, w_ref, b_ref, o_ref):
        a = a_ref[0, 0]
        views = (x10, x11, x12, x20, x21, x22)
        acc = jnp.zeros((bm, outc), _F32)
        for c in range(6):
            xb = jnp.maximum(views[c][0], 0.0)
            acc += a[c] * _dot(xb, w_ref[0, c])
        o_ref[0] = acc + b_ref[0]

    in_specs = [
        pl.BlockSpec((1, bm, f), lambda g, i, k=k: (3 * g + k, i, 0))
        for k in range(3)
    ] * 2
    in_specs += [
        pl.BlockSpec((1, 1, 6), lambda g, i: (g, 0, 0)),
        pl.BlockSpec((1, 6, f, outc), lambda g, i: (g, 0, 0, 0)),
        pl.BlockSpec((1, 1, outc), lambda g, i: (g, 0, 0)),
    ]
    return pl.pallas_call(
        body,
        grid=(2, n // bm),
        in_specs=in_specs,
        out_specs=pl.BlockSpec((1, bm, outc), lambda g, i: (g, i, 0)),
        out_shape=jax.ShapeDtypeStruct((2, n, outc), _F32),
    )(xs1, xs1, xs1, xs2, xs2, xs2, attw, w, bias)


def _final(emb, bm=256):
    n, outc = emb.shape[-2:]

    def body(a_ref, b_ref, o_ref):
        o_ref[...] = _dot_nt(a_ref[0], b_ref[0])

    return pl.pallas_call(
        body,
        grid=(n // bm,),
        in_specs=[
            pl.BlockSpec((1, bm, outc), lambda i: (0, i, 0)),
            pl.BlockSpec((1, n, outc), lambda i: (1, 0, 0)),
        ],
        out_specs=pl.BlockSpec((bm, n), lambda i: (i, 0)),
        out_shape=jax.ShapeDtypeStruct((n, n), _F32),
    )(emb, emb)


def kernel(mi_feature, d_feature, mm_g, mm_s, mm_h, dd_g, dd_s, dd_h,
           params_x, params_y):
    px, py = params_x, params_y
    x0 = jnp.stack([mi_feature.T, d_feature.T])
    adj6 = jnp.stack([mm_g, mm_s, mm_h, dd_g, dd_s, dd_h])

    wcat1 = jnp.stack([
        jnp.concatenate([w for (w, _) in p["gat1"]["heads"]], axis=1)
        for p in (px, py)])
    av1 = jnp.stack([a[:, 0] for p in (px, py)
                     for (_, a) in p["gat1"]["heads"]])[:, None, :]
    wout1 = jnp.stack([p["gat1"]["out"][0] for p in (px, py)])
    avo1 = jnp.stack([p["gat1"]["out"][1][:, 0] for p in (px, py)])[:, None, :]
    wcat2 = jnp.stack([
        jnp.concatenate([w for (w, _) in p["gat2"]["heads"]], axis=1)
        for p in (px, py)])
    av2 = jnp.stack([a[:, 0] for p in (px, py)
                     for (_, a) in p["gat2"]["heads"]])[:, None, :]
    wout2 = jnp.stack([p["gat2"]["out"][0] for p in (px, py)])
    avo2 = jnp.stack([p["gat2"]["out"][1][:, 0] for p in (px, py)])[:, None, :]

    ident = lambda g: g
    per_branch = lambda g: g // 3

    wh1 = _bmm(x0, wcat1, ident, ident, gb=2)
    hp1 = _attention(adj6, wh1, av1, nheads=2, feat=1024)
    whb = _bmm(hp1, wout1, ident, per_branch, gb=6)
    hpb = _attention(adj6, whb, avo1, nheads=1, feat=512)
    wh2 = _bmm(hpb, wcat2, ident, per_branch, gb=6)
    hp2 = _attention(adj6, wh2, av2, nheads=2, feat=512)
    whd = _bmm(hp2, wout2, ident, per_branch, gb=6)
    a_adj = _attention(adj6, whd, avo2, nheads=1, feat=1024)

    fc0wt = jnp.stack([p["fc0_W"].T for p in (px, py)])
    fc0b = jnp.stack([p["fc0_b"] for p in (px, py)])[:, None, :]
    li = _bmm(x0, fc0wt, ident, ident, gb=2,
              extras=[(fc0b, (1, 1, 1024), lambda g, m, n: (g, 0, n))],
              epilogue=lambda r, b: _lrelu(r + b, 0.25))
    h0 = li
    xs_layers = []
    for l in (1, 2):
        w_l = jnp.stack([p["convs"][l - 1] for p in (px, py)])
        theta = math.log(_LAMDA / l + 1.0)
        sup = _bmm(a_adj, li, ident, per_branch, gb=6,
                   extras=[(h0, (1, 256, 1024),
                            lambda g, m, n: (g // 3, m, n))],
                   epilogue=lambda r, h0b: (1.0 - _ALPHA) * r + _ALPHA * h0b)
        xs_l = _bmm(w_l, sup, per_branch, ident, gb=6, bn=512,
                    extras=[(sup, (1, 256, 512),
                             lambda g, m, n: (g, m, n)),
                            (li, (1, 256, 512),
                             lambda g, m, n: (g // 3, m, n))],
                    epilogue=lambda r, s_, l_, th=theta:
                        _lrelu(th * r + (1.0 - th) * s_ + l_, 0.25))
        xs_layers.append(xs_l)
        li = _mean3(xs_l)

    mu1 = _viewmeans(xs_layers[0])[:, 0, 0]
    mu2 = _viewmeans(xs_layers[1])[:, 0, 0]
    mu = jnp.concatenate([mu1.reshape(2, 3), mu2.reshape(2, 3)],
                         axis=1).reshape(2, 1, 6)
    fc1w = jnp.stack([p["fc1_W"] for p in (px, py)])
    fc1b = jnp.stack([p["fc1_b"] for p in (px, py)])[:, None, :]
    fc2w = jnp.stack([p["fc2_W"] for p in (px, py)])
    fc2b = jnp.stack([p["fc2_b"] for p in (px, py)])[:, None, :]
    attw = _view_mlp(mu, fc1w, fc1b, fc2w, fc2b)

    cnnw = jnp.stack([jnp.transpose(p["cnn_W"][..., 0], (1, 2, 0))
                      for p in (px, py)])
    cnnb = jnp.stack([p["cnn_b"] for p in (px, py)])[:, None, :]
    emb = _cnn(xs_layers[0], xs_layers[1], attw, cnnw, cnnb)
    return _final(emb)
